# baseline (speedup 1.0000x reference)
import functools

import jax
import jax.numpy as jnp
from jax import lax
from jax.experimental import pallas as pl
from jax.experimental.pallas import tpu as pltpu
from jax.experimental.pallas import tpu_sc as plsc

_B = 16384
_ND = 13
_NS = 26
_V = 100000
_D = 16

_NW = 32
_SW = _B // _NW
_SPC = 4
_RPC = _SPC * _NS
_C = _SW // _SPC
_NBUF = 4
_GRP = _NBUF * _SPC


def _fm_body(tab_hbm, idx_hbm, fm_hbm, idx_v, rows_v, stage_v, s0, s1, s2, s3):
    sems = (s0, s1, s2, s3)
    wid = lax.axis_index("c") * 16 + lax.axis_index("s")
    pltpu.sync_copy(idx_hbm.at[wid], idx_v)

    def start(c, b):
        pltpu.async_copy(tab_hbm.at[idx_v.at[c]], rows_v.at[b], sems[b])

    for b in range(_NBUF):
        start(b, b)

    def group(g, carry):
        for b in range(_NBUF):
            c = g * _NBUF + b
            pltpu.make_async_copy(tab_hbm.at[idx_v.at[c]], rows_v.at[b],
                                  sems[b]).wait()
            rbuf = rows_v.at[b]
            for s in range(_SPC):
                base = s * _NS
                e = rbuf[base]
                acc = e
                sq = e * e
                for j in range(1, _NS):
                    e = rbuf[base + j]
                    acc = acc + e
                    sq = sq + e * e
                stage_v[b * _SPC + s] = 0.5 * (acc * acc - sq)

            @pl.when(c + _NBUF < _C)
            def _():
                start(c + _NBUF, b)

        pltpu.sync_copy(stage_v, fm_hbm.at[pl.ds(wid * _SW + g * _GRP, _GRP)])
        return carry

    lax.fori_loop(0, _C // _NBUF, group, 0)


_fm_kernel = functools.partial(
    pl.kernel,
    mesh=plsc.VectorSubcoreMesh(core_axis_name="c", subcore_axis_name="s"),
    compiler_params=pltpu.CompilerParams(use_tc_tiling_on_sc=False),
    out_type=jax.ShapeDtypeStruct((_B, _D), jnp.float32),
    scratch_types=[
        pltpu.VMEM((_C, _RPC), jnp.int32),
        pltpu.VMEM((_NBUF, _RPC, _D), jnp.float32),
        pltpu.VMEM((_GRP, _D), jnp.float32),
        pltpu.SemaphoreType.DMA,
        pltpu.SemaphoreType.DMA,
        pltpu.SemaphoreType.DMA,
        pltpu.SemaphoreType.DMA,
    ],
)(_fm_body)


def _mlp_body(dense_ref, fm_ref, w1d_ref, w1f_ref, b1_ref, w2_ref, b2_ref,
              w3_ref, b3_ref, wot_ref, bo_ref, out_ref):
    h = jnp.dot(dense_ref[...], w1d_ref[...], preferred_element_type=jnp.float32)
    h = h + jnp.dot(fm_ref[...], w1f_ref[...], preferred_element_type=jnp.float32)
    h = jnp.maximum(h + b1_ref[...], 0.0)
    h = jnp.dot(h, w2_ref[...], preferred_element_type=jnp.float32) + b2_ref[...]
    h = jnp.maximum(h, 0.0)
    h = jnp.dot(h, w3_ref[...], preferred_element_type=jnp.float32) + b3_ref[...]
    h = jnp.maximum(h, 0.0)
    t = jnp.sum(h * wot_ref[...], axis=1, keepdims=True) + bo_ref[...]
    out_ref[...] = 1.0 / (1.0 + jnp.exp(-t))


def _mlp_call(dense, fm, w1d, w1f, b1n, w2, b2, w3, b3, wot, bo):
    bm = 4096
    full = lambda i: (0, 0)
    return pl.pallas_call(
        _mlp_body,
        grid=(_B // bm,),
        in_specs=[
            pl.BlockSpec((bm, _ND), lambda i: (i, 0)),
            pl.BlockSpec((bm, _D), lambda i: (i, 0)),
            pl.BlockSpec((_ND, 256), full),
            pl.BlockSpec((_D, 256), full),
            pl.BlockSpec((1, 256), full),
            pl.BlockSpec((256, 128), full),
            pl.BlockSpec((1, 128), full),
            pl.BlockSpec((128, 64), full),
            pl.BlockSpec((1, 64), full),
            pl.BlockSpec((1, 64), full),
            pl.BlockSpec((1, 1), full),
        ],
        out_specs=pl.BlockSpec((bm, 1), lambda i: (i, 0)),
        out_shape=jax.ShapeDtypeStruct((_B, 1), jnp.float32),
    )(dense, fm, w1d, w1f, b1n, w2, b2, w3, b3, wot, bo)


def kernel(inputs, tables, gamma, beta, moving_mean, moving_var,
           W1, b1, W2, b2, W3, b3, Wo, bo):
    dense = inputs[:, :_ND]
    idx = inputs[:, _ND:].astype(jnp.int32)
    flat = idx + (jnp.arange(_NS, dtype=jnp.int32) * _V)[None, :]
    idx3 = flat.reshape(_NW, _C, _RPC)
    tab2 = tables.reshape(_NS * _V, _D)

    fm = _fm_kernel(tab2, idx3)

    a = gamma / jnp.sqrt(moving_var + 1e-3)
    c0 = beta - moving_mean * a
    w1n = a[:, None] * W1
    b1n = (b1 + c0 @ W1).reshape(1, 256)
    w1d = w1n[:_ND]
    w1f = w1n[_ND:]

    return _mlp_call(dense, fm, w1d, w1f, b1n,
                     W2, b2.reshape(1, 128), W3, b3.reshape(1, 64),
                     Wo.reshape(1, 64), bo.reshape(1, 1))
